# Initial kernel scaffold; baseline (speedup 1.0000x reference)
import jax
import jax.numpy as jnp
from jax.experimental import pallas as pl


def kernel(y, x, t2v_weight, t2v_bias, y_emb_W, y_emb_b, var_emb_table, given_emb_table):
    raise NotImplementedError("write your pallas kernel here")


# speedup vs baseline: 5.8784x; 5.8784x over previous
import jax
import jax.numpy as jnp
from jax.experimental import pallas as pl

BS, LENGTH, D_Y, D_X, D_MODEL = 8, 512, 8, 7, 512
T2V_IN = D_X + 1
T2V_K = D_MODEL // T2V_IN


def _tc_body(xc_ref, y_ref, e_ref, w_ref, b_ref, w0_ref, w1_ref, c_ref,
             tab_ref, out_ref, vemb_ref, vidx_ref):
    xc = xc_ref[0]
    xce = jax.lax.dot(xc, e_ref[...], precision=jax.lax.Precision.HIGHEST)
    a = xce * w_ref[...] + b_ref[...]
    ids = jax.lax.broadcasted_iota(jnp.int32, (LENGTH, D_MODEL), 1)
    s = jnp.where((ids & (T2V_K - 1)) == 0, a, jnp.sin(a))
    t = jax.lax.dot(s, w1_ref[...],
                    precision=jax.lax.Precision.HIGHEST) + c_ref[...]
    yb = y_ref[0]
    for v in range(D_Y):
        yv = yb[:, v:v + 1]
        out_ref[0, v * LENGTH:(v + 1) * LENGTH, :] = t + yv * w0_ref[...]
        row = tab_ref[v:v + 1, :]
        vemb_ref[0, v * LENGTH:(v + 1) * LENGTH, :] = jnp.broadcast_to(
            row, (LENGTH, D_MODEL))
    vidx_ref[0] = (jax.lax.broadcasted_iota(jnp.int32, (1, D_Y * LENGTH), 1)
                   >> 9)


def kernel(y, x, t2v_weight, t2v_bias, y_emb_W, y_emb_b, var_emb_table,
           given_emb_table):
    local_pos = jnp.broadcast_to(
        (jnp.arange(LENGTH, dtype=jnp.float32) / LENGTH)[None, :, None],
        (BS, LENGTH, 1))
    xc = jnp.concatenate([x, local_pos], axis=-1)
    e = jnp.repeat(jnp.eye(T2V_IN, dtype=jnp.float32), T2V_K, axis=1)
    wrow = t2v_weight.reshape(1, D_MODEL)
    brow = t2v_bias.reshape(1, D_MODEL)
    w0 = y_emb_W[0:1]
    w1 = y_emb_W[1:]
    c = (y_emb_b + given_emb_table[1])[None]

    n = D_Y * LENGTH
    grid = (BS,)
    val_time, var_emb, var_idx3 = pl.pallas_call(
        _tc_body,
        grid=grid,
        in_specs=[
            pl.BlockSpec((1, LENGTH, T2V_IN), lambda b: (b, 0, 0)),
            pl.BlockSpec((1, LENGTH, D_Y), lambda b: (b, 0, 0)),
            pl.BlockSpec((T2V_IN, D_MODEL), lambda b: (0, 0)),
            pl.BlockSpec((1, D_MODEL), lambda b: (0, 0)),
            pl.BlockSpec((1, D_MODEL), lambda b: (0, 0)),
            pl.BlockSpec((1, D_MODEL), lambda b: (0, 0)),
            pl.BlockSpec((D_MODEL, D_MODEL), lambda b: (0, 0)),
            pl.BlockSpec((1, D_MODEL), lambda b: (0, 0)),
            pl.BlockSpec((D_Y, D_MODEL), lambda b: (0, 0)),
        ],
        out_specs=[
            pl.BlockSpec((1, n, D_MODEL), lambda b: (b, 0, 0)),
            pl.BlockSpec((1, n, D_MODEL), lambda b: (b, 0, 0)),
            pl.BlockSpec((1, 1, n), lambda b: (b, 0, 0)),
        ],
        out_shape=[
            jax.ShapeDtypeStruct((BS, n, D_MODEL), jnp.float32),
            jax.ShapeDtypeStruct((BS, n, D_MODEL), jnp.float32),
            jax.ShapeDtypeStruct((BS, 1, n), jnp.int32),
        ],
    )(xc, y, e, wrow, brow, w0, w1, c, var_emb_table)
    return val_time, var_emb, var_idx3.reshape(BS, n)
